# Initial kernel scaffold; baseline (speedup 1.0000x reference)
import functools

import jax
import jax.numpy as jnp
from jax import lax
from jax.experimental import pallas as pl
from jax.experimental.pallas import tpu as pltpu
from jax.experimental.pallas import tpu_sc as plsc

SB = 256
EC = 64
NW = 32
BPL = 208


def _tc_layer(xp, W, AsP, AdP):
  Npad, Din = xp.shape
  HC = W.shape[1]
  R = 512
  G = Npad // R

  def body(x_ref, w_ref, as_ref, ad_ref, h_ref, asrc_ref, adst_ref, gmax_ref):
    i = pl.program_id(0)
    h = jnp.dot(x_ref[...], w_ref[...], preferred_element_type=jnp.float32)
    h_ref[...] = h
    a_s = jnp.dot(h, as_ref[...], preferred_element_type=jnp.float32)
    a_d = jnp.dot(h, ad_ref[...], preferred_element_type=jnp.float32)
    asrc_ref[...] = a_s
    adst_ref[...] = a_d
    m = jnp.broadcast_to(jnp.max(a_s, axis=0, keepdims=True), (8, 16))

    @pl.when(i == 0)
    def _():
      gmax_ref[...] = m

    @pl.when(i != 0)
    def _():
      gmax_ref[...] = jnp.maximum(gmax_ref[...], m)

  return pl.pallas_call(
      body,
      grid=(G,),
      in_specs=[
          pl.BlockSpec((R, Din), lambda i: (i, 0)),
          pl.BlockSpec((Din, HC), lambda i: (0, 0)),
          pl.BlockSpec((HC, 16), lambda i: (0, 0)),
          pl.BlockSpec((HC, 16), lambda i: (0, 0)),
      ],
      out_specs=[
          pl.BlockSpec((R, HC), lambda i: (i, 0)),
          pl.BlockSpec((R, 16), lambda i: (i, 0)),
          pl.BlockSpec((R, 16), lambda i: (i, 0)),
          pl.BlockSpec((8, 16), lambda i: (0, 0)),
      ],
      out_shape=[
          jax.ShapeDtypeStruct((Npad, HC), jnp.float32),
          jax.ShapeDtypeStruct((Npad, 16), jnp.float32),
          jax.ShapeDtypeStruct((Npad, 16), jnp.float32),
          jax.ShapeDtypeStruct((8, 16), jnp.float32),
      ],
  )(xp, W, AsP, AdP)


def _sc_layer(h, asrcP, adstP, gmax, bias, ssrc, sdst, bp, H, C, n_real):
  HC = H * C
  Npad = h.shape[0]
  NB = Npad // SB
  ITER = (NB + NW - 1) // NW
  mesh = plsc.VectorSubcoreMesh(core_axis_name="c", subcore_axis_name="s")

  @functools.partial(
      pl.kernel,
      mesh=mesh,
      out_type=jax.ShapeDtypeStruct((Npad, HC), jnp.float32),
      scratch_types=[
          pltpu.VMEM((BPL,), jnp.int32),
          pltpu.VMEM((16,), jnp.float32),
          pltpu.VMEM((HC,), jnp.float32),
          pltpu.VMEM((SB, 16), jnp.float32),
          pltpu.VMEM((SB, H, 16), jnp.float32),
          pltpu.VMEM((SB, 16), jnp.float32),
          pltpu.VMEM((SB, HC), jnp.float32),
          pltpu.VMEM((EC,), jnp.int32),
          pltpu.VMEM((EC,), jnp.int32),
          pltpu.VMEM((EC, 16), jnp.float32),
          pltpu.VMEM((EC, HC), jnp.float32),
          pltpu.VMEM((EC, 16), jnp.float32),
          pltpu.VMEM((EC,), jnp.int32),
          pltpu.SemaphoreType.DMA,
      ],
  )
  def k(h_hbm, asrc_hbm, adst_hbm, gmax_hbm, bias_hbm, ssrc_hbm, sdst_hbm,
        bp_hbm, out_hbm, bp_v, gmax_v, bias_v, adst_blk, s_spread, invs_blk,
        acc, src_v, dst_v, asrc_g, h_g, alpha_buf, dl_buf, sem):
    wid = lax.axis_index("s") * 2 + lax.axis_index("c")
    pltpu.sync_copy(bp_hbm, bp_v)
    pltpu.sync_copy(gmax_hbm, gmax_v)
    pltpu.sync_copy(bias_hbm, bias_v)
    iota = lax.broadcasted_iota(jnp.int32, (16,), 0)
    zf = jnp.zeros((16,), jnp.float32)

    def edge_logits(g, base, b0, e_lo, e_hi, hh):
      ge = base + g * 16 + iota
      valid = (ge >= e_lo) & (ge < e_hi)
      dst16 = dst_v[pl.ds(g * 16, 16)]
      dl = jnp.clip(dst16 - b0, 0, SB - 1)
      eidx = g * 16 + iota
      hsp = jnp.full((16,), hh, jnp.int32)
      av = plsc.load_gather(asrc_g, [eidx, hsp])
      dv = plsc.load_gather(adst_blk, [dl, hsp])
      gm = gmax_v[hh]
      e16 = av + dv
      e16 = jnp.where(e16 >= 0, e16, 0.2 * e16)
      c16 = gm + dv
      c16 = jnp.where(c16 >= 0, c16, 0.2 * c16)
      ex = jnp.exp(e16 - c16)
      return dl, valid, ex, hsp

    def do_block(blk):
      b0 = blk * SB
      e_lo = bp_v[blk]
      e_hi = bp_v[blk + 1]
      abase = (e_lo // 8) * 8
      nch = (e_hi - abase + (EC - 1)) // EC
      pltpu.sync_copy(adst_hbm.at[pl.ds(b0, SB)], adst_blk)

      def zbody(i, carry):
        for hh in range(H):
          s_spread[i, hh, pl.ds(0, 16)] = zf
        for j in range(HC // 16):
          acc[i, pl.ds(j * 16, 16)] = zf
        return carry

      lax.fori_loop(0, SB, zbody, 0)

      def chunkA(ci, carry):
        base = abase + ci * EC
        pltpu.sync_copy(ssrc_hbm.at[pl.ds(base, EC)], src_v)
        pltpu.sync_copy(sdst_hbm.at[pl.ds(base, EC)], dst_v)
        pltpu.async_copy(asrc_hbm.at[src_v], asrc_g, sem).wait()
        for g in range(EC // 16):
          for hh in range(H):
            dl, valid, ex, hsp = edge_logits(g, base, b0, e_lo, e_hi, hh)
            plsc.addupdate_scatter(s_spread, [dl, hsp, iota], ex, mask=valid)
        return carry

      lax.fori_loop(0, nch, chunkA, 0)

      def rbody(i, carry):
        for hh in range(H):
          v = s_spread[i, hh, pl.ds(0, 16)]
          r = jnp.sum(v)
          invs_blk[i, hh] = 1.0 / (r + 1e-16)
        return carry

      lax.fori_loop(0, SB, rbody, 0)

      def chunkB(ci, carry):
        base = abase + ci * EC
        pltpu.sync_copy(ssrc_hbm.at[pl.ds(base, EC)], src_v)
        pltpu.sync_copy(sdst_hbm.at[pl.ds(base, EC)], dst_v)
        pltpu.async_copy(asrc_hbm.at[src_v], asrc_g, sem).wait()
        pltpu.async_copy(h_hbm.at[src_v], h_g, sem).wait()
        for g in range(EC // 16):
          dl0 = None
          for hh in range(H):
            dl, valid, ex, hsp = edge_logits(g, base, b0, e_lo, e_hi, hh)
            iv = plsc.load_gather(invs_blk, [dl, hsp])
            alpha = jnp.where(valid, ex * iv, 0.0)
            plsc.store_scatter(alpha_buf, [g * 16 + iota, hsp], alpha)
            dl0 = dl
          dl_buf[pl.ds(g * 16, 16)] = dl0

        def ebody(e, carry2):
          d = dl_buf[e]
          for hh in range(H):
            a = alpha_buf[e, hh]
            for cc in range(C // 16):
              off = hh * C + cc * 16
              v = h_g[e, pl.ds(off, 16)] * a
              plsc.addupdate(acc.at[d, pl.ds(off, 16)], v)
          return carry2

        lax.fori_loop(0, EC, ebody, 0)
        return carry

      lax.fori_loop(0, nch, chunkB, 0)

      def fbody(i, carry):
        rmask = jnp.where(b0 + i < n_real, 1.0, 0.0)
        for j in range(HC // 16):
          v = acc[i, pl.ds(j * 16, 16)] + bias_v[pl.ds(j * 16, 16)]
          v = jnp.where(v > 0, v, jnp.exp(v) - 1.0)
          acc[i, pl.ds(j * 16, 16)] = v * rmask
        return carry

      lax.fori_loop(0, SB, fbody, 0)
      pltpu.sync_copy(acc, out_hbm.at[pl.ds(b0, SB)])

    def wbody(it, carry):
      blk = it * NW + wid

      @pl.when(blk < NB)
      def _():
        do_block(blk)

      return carry

    lax.fori_loop(0, ITER, wbody, 0)

  return k(h, asrcP, adstP, gmax, bias, ssrc, sdst, bp)


def _tc_head(x4, fw1, fb1, fw2, fb2, fw3p, fb3p, n):
  def body(x_ref, w1, b1, w2, b2, w3, b3, o_ref):
    g = jnp.sum(x_ref[...], axis=0, keepdims=True) * (1.0 / n)
    z = jnp.maximum(jnp.dot(g, w1[...], preferred_element_type=jnp.float32)
                    + b1[...], 0.0)
    z = jnp.maximum(jnp.dot(z, w2[...], preferred_element_type=jnp.float32)
                    + b2[...], 0.0)
    z = jax.nn.sigmoid(jnp.dot(z, w3[...], preferred_element_type=jnp.float32)
                       + b3[...])
    o_ref[...] = z

  return pl.pallas_call(
      body,
      out_shape=jax.ShapeDtypeStruct((1, 128), jnp.float32),
  )(x4, fw1, fb1, fw2, fb2, fw3p, fb3p)


def _headmat(a):
  Hh, Cc = a.shape
  M = jnp.zeros((Hh * Cc, 16), jnp.float32)
  rows = jnp.arange(Hh * Cc)
  cols = jnp.repeat(jnp.arange(Hh), Cc)
  return M.at[rows, cols].set(a.reshape(-1))


def kernel(x, edge_index, W1, as1, ad1, b1, W2, as2, ad2, b2, W3, as3, ad3,
           b3, fw1, fb1, fw2, fb2, fw3, fb3):
  N = x.shape[0]
  E = edge_index.shape[1]
  loop = jnp.arange(N, dtype=jnp.int32)
  src = jnp.concatenate([edge_index[0].astype(jnp.int32), loop])
  dst = jnp.concatenate([edge_index[1].astype(jnp.int32), loop])
  sdst, ssrc = lax.sort_key_val(dst, src)

  Etot = E + N
  Npad = ((N + SB - 1) // SB) * SB
  NB = Npad // SB
  Epad = ((Etot + 2 * EC) // EC) * EC
  ssrc = jnp.pad(ssrc, (0, Epad - Etot))
  sdst = jnp.pad(sdst, (0, Epad - Etot), constant_values=Npad)
  bvals = jnp.arange(NB + 1, dtype=jnp.int32) * SB
  bp = jnp.searchsorted(sdst, bvals).astype(jnp.int32)
  bp = jnp.pad(bp, (0, BPL - (NB + 1)), constant_values=Etot)

  xp = jnp.pad(x, ((0, Npad - N), (0, 128 - x.shape[1])))
  W1p = jnp.pad(W1, ((0, 128 - W1.shape[0]), (0, 0)))

  h1, s1, d1, g1 = _tc_layer(xp, W1p, _headmat(as1), _headmat(ad1))
  x2 = _sc_layer(h1, s1, d1, g1[0], b1, ssrc, sdst, bp, 4, 64, N)
  h2, s2, d2, g2 = _tc_layer(x2, W2, _headmat(as2), _headmat(ad2))
  x3 = _sc_layer(h2, s2, d2, g2[0], b2, ssrc, sdst, bp, 4, 64, N)
  h3, s3, d3, g3 = _tc_layer(x3, W3, _headmat(as3), _headmat(ad3))
  x4 = _sc_layer(h3, s3, d3, g3[0], b3, ssrc, sdst, bp, 1, 32, N)

  fw3p = jnp.pad(fw3, ((0, 0), (0, 127)))
  fb3p = jnp.pad(fb3, (0, 127)).reshape(1, 128)
  out = _tc_head(x4, fw1, fb1.reshape(1, -1), fw2, fb2.reshape(1, -1),
                 fw3p, fb3p, N)
  return out[0, 0:1]


# speedup vs baseline: 17.3775x; 17.3775x over previous
import functools

import jax
import jax.numpy as jnp
from jax import lax
from jax.experimental import pallas as pl
from jax.experimental.pallas import tpu as pltpu
from jax.experimental.pallas import tpu_sc as plsc

SB = 256
EC = 64
NW = 32
BPL = 224


def _tc_layer(xp, W, AsP, AdP):
  Npad, Din = xp.shape
  HC = W.shape[1]
  R = 512
  G = Npad // R

  def body(x_ref, w_ref, as_ref, ad_ref, h_ref, asrc_ref, adst_ref, gmax_ref):
    i = pl.program_id(0)
    h = jnp.dot(x_ref[...], w_ref[...], preferred_element_type=jnp.float32)
    h_ref[...] = h
    a_s = jnp.dot(h, as_ref[...], preferred_element_type=jnp.float32)
    a_d = jnp.dot(h, ad_ref[...], preferred_element_type=jnp.float32)
    asrc_ref[...] = a_s
    adst_ref[...] = a_d
    m = jnp.broadcast_to(jnp.max(a_s, axis=0, keepdims=True), (8, 16))

    @pl.when(i == 0)
    def _():
      gmax_ref[...] = m

    @pl.when(i != 0)
    def _():
      gmax_ref[...] = jnp.maximum(gmax_ref[...], m)

  return pl.pallas_call(
      body,
      grid=(G,),
      in_specs=[
          pl.BlockSpec((R, Din), lambda i: (i, 0)),
          pl.BlockSpec((Din, HC), lambda i: (0, 0)),
          pl.BlockSpec((HC, 16), lambda i: (0, 0)),
          pl.BlockSpec((HC, 16), lambda i: (0, 0)),
      ],
      out_specs=[
          pl.BlockSpec((R, HC), lambda i: (i, 0)),
          pl.BlockSpec((R, 16), lambda i: (i, 0)),
          pl.BlockSpec((R, 16), lambda i: (i, 0)),
          pl.BlockSpec((8, 16), lambda i: (0, 0)),
      ],
      out_shape=[
          jax.ShapeDtypeStruct((Npad, HC), jnp.float32),
          jax.ShapeDtypeStruct((Npad, 16), jnp.float32),
          jax.ShapeDtypeStruct((Npad, 16), jnp.float32),
          jax.ShapeDtypeStruct((8, 16), jnp.float32),
      ],
  )(xp, W, AsP, AdP)


def _sc_layer(h, asrcP, adstP, gmax, bias, ssrc, sdst, bp, H, C, n_real):
  HC = H * C
  Npad = h.shape[0]
  NB = Npad // SB
  ITER = (NB + NW - 1) // NW
  mesh = plsc.VectorSubcoreMesh(core_axis_name="c", subcore_axis_name="s")

  @functools.partial(
      pl.kernel,
      mesh=mesh,
      out_type=jax.ShapeDtypeStruct((Npad, HC), jnp.float32),
      compiler_params=pltpu.CompilerParams(use_tc_tiling_on_sc=False),
      scratch_types=[
          pltpu.VMEM((BPL,), jnp.int32),
          pltpu.VMEM((16,), jnp.float32),
          pltpu.VMEM((HC,), jnp.float32),
          pltpu.VMEM((SB, 16), jnp.float32),
          pltpu.VMEM((SB, 16), jnp.float32),
          pltpu.VMEM((SB, HC), jnp.float32),
          pltpu.VMEM((EC,), jnp.int32),
          pltpu.VMEM((EC + 16,), jnp.int32),
          pltpu.VMEM((EC, 16), jnp.float32),
          pltpu.VMEM((EC, HC), jnp.float32),
          pltpu.SemaphoreType.DMA,
      ],
  )
  def k(h_hbm, asrc_hbm, adst_hbm, gmax_hbm, bias_hbm, ssrc_hbm, sdst_hbm,
        bp_hbm, out_hbm, bp_v, gmax_v, bias_v, adst_blk, s_blk, acc,
        src_v, dst_v, asrc_g, h_g, sem):
    wid = lax.axis_index("s") * 2 + lax.axis_index("c")
    pltpu.sync_copy(bp_hbm, bp_v)
    pltpu.sync_copy(gmax_hbm, gmax_v)
    pltpu.sync_copy(bias_hbm, bias_v)
    zf = jnp.zeros((16,), jnp.float32)
    gv = gmax_v[pl.ds(0, 16)]

    def do_block(blk):
      b0 = blk * SB
      bpv16 = bp_v[pl.ds(blk, 16)]
      e_lo = bpv16[0]
      e_hi = bpv16[1]
      abase = (e_lo // 8) * 8
      nch = (e_hi - abase + (EC - 1)) // EC
      pltpu.sync_copy(adst_hbm.at[pl.ds(b0, SB)], adst_blk)

      def zbody(i, carry):
        s_blk[i, pl.ds(0, 16)] = zf
        for j in range(HC // 16):
          acc[i, pl.ds(j * 16, 16)] = zf
        return carry

      lax.fori_loop(0, SB, zbody, 0)

      def chunk(ci, carry):
        base = abase + ci * EC
        pltpu.sync_copy(ssrc_hbm.at[pl.ds(base, EC)], src_v)
        pltpu.sync_copy(sdst_hbm.at[pl.ds(base, EC)], dst_v.at[pl.ds(0, EC)])
        pltpu.async_copy(asrc_hbm.at[src_v], asrc_g, sem).wait()
        pltpu.async_copy(h_hbm.at[src_v], h_g, sem).wait()

        def ebody(e, carry2):
          ge = base + e
          valid = (ge >= e_lo) & (ge < e_hi)
          d = dst_v[pl.ds(e, 16)][0]
          dl = jnp.clip(d - b0, 0, SB - 1)
          arow = asrc_g[e, pl.ds(0, 16)]
          drow = adst_blk[dl, pl.ds(0, 16)]
          ev = arow + drow
          ev = jnp.where(ev >= 0, ev, 0.2 * ev)
          cv = gv + drow
          cv = jnp.where(cv >= 0, cv, 0.2 * cv)
          exv = jnp.exp(ev - cv) * jnp.where(valid, 1.0, 0.0)
          plsc.addupdate(s_blk.at[dl, pl.ds(0, 16)], exv)
          for hh in range(H):
            a = exv[hh]
            for cc in range(C // 16):
              off = hh * C + cc * 16
              v = h_g[e, pl.ds(off, 16)] * a
              plsc.addupdate(acc.at[dl, pl.ds(off, 16)], v)
          return carry2

        lax.fori_loop(0, EC, ebody, 0)
        return carry

      lax.fori_loop(0, nch, chunk, 0)

      def fbody(i, carry):
        rmask = jnp.where(b0 + i < n_real, 1.0, 0.0)
        inv = 1.0 / (s_blk[i, pl.ds(0, 16)] + 1e-16)
        for hh in range(H):
          ivs = inv[hh]
          for cc in range(C // 16):
            off = hh * C + cc * 16
            v = acc[i, pl.ds(off, 16)] * ivs + bias_v[pl.ds(off, 16)]
            v = jnp.where(v > 0, v, jnp.exp(v) - 1.0)
            acc[i, pl.ds(off, 16)] = v * rmask
        return carry

      lax.fori_loop(0, SB, fbody, 0)
      pltpu.sync_copy(acc, out_hbm.at[pl.ds(b0, SB)])

    def wbody(it, carry):
      blk = it * NW + wid

      @pl.when(blk < NB)
      def _():
        do_block(blk)

      return carry

    lax.fori_loop(0, ITER, wbody, 0)

  return k(h, asrcP, adstP, gmax, bias, ssrc, sdst, bp)


def _tc_head(x4, fw1, fb1, fw2, fb2, fw3p, fb3p, n):
  def body(x_ref, w1, b1, w2, b2, w3, b3, o_ref):
    g = jnp.sum(x_ref[...], axis=0, keepdims=True) * (1.0 / n)
    z = jnp.maximum(jnp.dot(g, w1[...], preferred_element_type=jnp.float32)
                    + b1[...], 0.0)
    z = jnp.maximum(jnp.dot(z, w2[...], preferred_element_type=jnp.float32)
                    + b2[...], 0.0)
    z = jax.nn.sigmoid(jnp.dot(z, w3[...], preferred_element_type=jnp.float32)
                       + b3[...])
    o_ref[...] = z

  return pl.pallas_call(
      body,
      out_shape=jax.ShapeDtypeStruct((1, 128), jnp.float32),
  )(x4, fw1, fb1, fw2, fb2, fw3p, fb3p)


def _headmat(a):
  Hh, Cc = a.shape
  M = jnp.zeros((Hh * Cc, 16), jnp.float32)
  rows = jnp.arange(Hh * Cc)
  cols = jnp.repeat(jnp.arange(Hh), Cc)
  return M.at[rows, cols].set(a.reshape(-1))


def kernel(x, edge_index, W1, as1, ad1, b1, W2, as2, ad2, b2, W3, as3, ad3,
           b3, fw1, fb1, fw2, fb2, fw3, fb3):
  N = x.shape[0]
  E = edge_index.shape[1]
  loop = jnp.arange(N, dtype=jnp.int32)
  src = jnp.concatenate([edge_index[0].astype(jnp.int32), loop])
  dst = jnp.concatenate([edge_index[1].astype(jnp.int32), loop])
  sdst, ssrc = lax.sort_key_val(dst, src)

  Etot = E + N
  Npad = ((N + SB - 1) // SB) * SB
  NB = Npad // SB
  Epad = ((Etot + 2 * EC) // EC) * EC
  ssrc = jnp.pad(ssrc, (0, Epad - Etot))
  sdst = jnp.pad(sdst, (0, Epad - Etot), constant_values=Npad)
  bvals = jnp.arange(NB + 1, dtype=jnp.int32) * SB
  bp = jnp.searchsorted(sdst, bvals).astype(jnp.int32)
  bp = jnp.pad(bp, (0, BPL - (NB + 1)), constant_values=Etot)

  xp = jnp.pad(x, ((0, Npad - N), (0, 128 - x.shape[1])))
  W1p = jnp.pad(W1, ((0, 128 - W1.shape[0]), (0, 0)))

  h1, s1, d1, g1 = _tc_layer(xp, W1p, _headmat(as1), _headmat(ad1))
  x2 = _sc_layer(h1, s1, d1, g1[0], b1, ssrc, sdst, bp, 4, 64, N)
  h2, s2, d2, g2 = _tc_layer(x2, W2, _headmat(as2), _headmat(ad2))
  x3 = _sc_layer(h2, s2, d2, g2[0], b2, ssrc, sdst, bp, 4, 64, N)
  h3, s3, d3, g3 = _tc_layer(x3, W3, _headmat(as3), _headmat(ad3))
  x4 = _sc_layer(h3, s3, d3, g3[0], b3, ssrc, sdst, bp, 1, 32, N)

  fw3p = jnp.pad(fw3, ((0, 0), (0, 127)))
  fb3p = jnp.pad(fb3, (0, 127)).reshape(1, 128)
  out = _tc_head(x4, fw1, fb1.reshape(1, -1), fw2, fb2.reshape(1, -1),
                 fw3p, fb3p, N)
  return out[0, 0:1]


# speedup vs baseline: 19.1985x; 1.1048x over previous
import functools

import jax
import jax.numpy as jnp
from jax import lax
from jax.experimental import pallas as pl
from jax.experimental.pallas import tpu as pltpu
from jax.experimental.pallas import tpu_sc as plsc

SB = 256
EC = 64
NW = 32
BPL = 224


def _tc_layer(xp, W, AsP, AdP):
  Npad, Din = xp.shape
  HC = W.shape[1]
  R = 512
  G = Npad // R

  def body(x_ref, w_ref, as_ref, ad_ref, h_ref, asrc_ref, adst_ref, gmax_ref):
    i = pl.program_id(0)
    h = jnp.dot(x_ref[...], w_ref[...], preferred_element_type=jnp.float32)
    h_ref[...] = h
    a_s = jnp.dot(h, as_ref[...], preferred_element_type=jnp.float32)
    a_d = jnp.dot(h, ad_ref[...], preferred_element_type=jnp.float32)
    asrc_ref[...] = a_s
    adst_ref[...] = a_d
    m = jnp.broadcast_to(jnp.max(a_s, axis=0, keepdims=True), (8, 16))

    @pl.when(i == 0)
    def _():
      gmax_ref[...] = m

    @pl.when(i != 0)
    def _():
      gmax_ref[...] = jnp.maximum(gmax_ref[...], m)

  return pl.pallas_call(
      body,
      grid=(G,),
      in_specs=[
          pl.BlockSpec((R, Din), lambda i: (i, 0)),
          pl.BlockSpec((Din, HC), lambda i: (0, 0)),
          pl.BlockSpec((HC, 16), lambda i: (0, 0)),
          pl.BlockSpec((HC, 16), lambda i: (0, 0)),
      ],
      out_specs=[
          pl.BlockSpec((R, HC), lambda i: (i, 0)),
          pl.BlockSpec((R, 16), lambda i: (i, 0)),
          pl.BlockSpec((R, 16), lambda i: (i, 0)),
          pl.BlockSpec((8, 16), lambda i: (0, 0)),
      ],
      out_shape=[
          jax.ShapeDtypeStruct((Npad, HC), jnp.float32),
          jax.ShapeDtypeStruct((Npad, 16), jnp.float32),
          jax.ShapeDtypeStruct((Npad, 16), jnp.float32),
          jax.ShapeDtypeStruct((8, 16), jnp.float32),
      ],
  )(xp, W, AsP, AdP)


def _sc_layer(h, asrcP, adstP, gmax, bias, ssrc, sdst, bp, H, C, n_real):
  HC = H * C
  Npad = h.shape[0]
  NB = Npad // SB
  ITER = (NB + NW - 1) // NW
  mesh = plsc.VectorSubcoreMesh(core_axis_name="c", subcore_axis_name="s")

  @functools.partial(
      pl.kernel,
      mesh=mesh,
      out_type=jax.ShapeDtypeStruct((Npad, HC), jnp.float32),
      compiler_params=pltpu.CompilerParams(use_tc_tiling_on_sc=False),
      scratch_types=[
          pltpu.VMEM((BPL,), jnp.int32),
          pltpu.VMEM((16,), jnp.float32),
          pltpu.VMEM((HC,), jnp.float32),
          pltpu.VMEM((SB, 16), jnp.float32),
          pltpu.VMEM((SB, 16), jnp.float32),
          pltpu.VMEM((SB, HC), jnp.float32),
          pltpu.VMEM((EC,), jnp.int32),
          pltpu.VMEM((EC + 16,), jnp.int32),
          pltpu.VMEM((EC, 16), jnp.float32),
          pltpu.VMEM((EC, HC), jnp.float32),
          pltpu.SemaphoreType.DMA,
      ],
  )
  def k(h_hbm, asrc_hbm, adst_hbm, gmax_hbm, bias_hbm, ssrc_hbm, sdst_hbm,
        bp_hbm, out_hbm, bp_v, gmax_v, bias_v, adst_blk, s_blk, acc,
        src_v, dst_v, asrc_g, h_g, sem):
    wid = lax.axis_index("s") * 2 + lax.axis_index("c")
    pltpu.sync_copy(bp_hbm, bp_v)
    pltpu.sync_copy(gmax_hbm, gmax_v)
    pltpu.sync_copy(bias_hbm, bias_v)
    zf = jnp.zeros((16,), jnp.float32)
    gv = gmax_v[pl.ds(0, 16)]

    def do_block(blk):
      b0 = blk * SB
      bpv16 = bp_v[pl.ds(blk, 16)]
      e_lo = bpv16[0]
      e_hi = bpv16[1]
      abase = (e_lo // 8) * 8
      nch = (e_hi - abase + (EC - 1)) // EC
      pltpu.sync_copy(adst_hbm.at[pl.ds(b0, SB)], adst_blk)

      def zbody(i, carry):
        s_blk[i, pl.ds(0, 16)] = zf
        for j in range(HC // 16):
          acc[i, pl.ds(j * 16, 16)] = zf
        return carry

      lax.fori_loop(0, SB, zbody, 0)

      iota = lax.broadcasted_iota(jnp.int32, (16,), 0)

      def chunk(ci, carry):
        base = abase + ci * EC
        pltpu.sync_copy(ssrc_hbm.at[pl.ds(base, EC)], src_v)
        pltpu.sync_copy(sdst_hbm.at[pl.ds(base, EC)], dst_v.at[pl.ds(0, EC)])
        pltpu.async_copy(asrc_hbm.at[src_v], asrc_g, sem).wait()
        pltpu.async_copy(h_hbm.at[src_v], h_g, sem).wait()

        def gbody(g, carry2):
          e0 = g * 16
          dvec = dst_v[pl.ds(e0, 16)]
          dlv = jnp.clip(dvec - b0, 0, SB - 1)
          gev = base + e0 + iota
          validf = jnp.where((gev >= e_lo) & (gev < e_hi), 1.0, 0.0)
          for j in range(16):
            e = e0 + j
            dl = dlv[j]
            arow = asrc_g[e, pl.ds(0, 16)]
            drow = adst_blk[dl, pl.ds(0, 16)]
            ev = arow + drow
            ev = jnp.where(ev >= 0, ev, 0.2 * ev)
            cv = gv + drow
            cv = jnp.where(cv >= 0, cv, 0.2 * cv)
            exv = jnp.exp(ev - cv) * validf[j]
            plsc.addupdate(s_blk.at[dl, pl.ds(0, 16)], exv)
            for hh in range(H):
              a = exv[hh]
              for cc in range(C // 16):
                off = hh * C + cc * 16
                v = h_g[e, pl.ds(off, 16)] * a
                plsc.addupdate(acc.at[dl, pl.ds(off, 16)], v)
          return carry2

        lax.fori_loop(0, EC // 16, gbody, 0)
        return carry

      lax.fori_loop(0, nch, chunk, 0)

      def fbody(i, carry):
        rmask = jnp.where(b0 + i < n_real, 1.0, 0.0)
        inv = 1.0 / (s_blk[i, pl.ds(0, 16)] + 1e-16)
        for hh in range(H):
          ivs = inv[hh]
          for cc in range(C // 16):
            off = hh * C + cc * 16
            v = acc[i, pl.ds(off, 16)] * ivs + bias_v[pl.ds(off, 16)]
            v = jnp.where(v > 0, v, jnp.exp(v) - 1.0)
            acc[i, pl.ds(off, 16)] = v * rmask
        return carry

      lax.fori_loop(0, SB, fbody, 0)
      pltpu.sync_copy(acc, out_hbm.at[pl.ds(b0, SB)])

    def wbody(it, carry):
      blk = it * NW + wid

      @pl.when(blk < NB)
      def _():
        do_block(blk)

      return carry

    lax.fori_loop(0, ITER, wbody, 0)

  return k(h, asrcP, adstP, gmax, bias, ssrc, sdst, bp)


def _tc_head(x4, fw1, fb1, fw2, fb2, fw3p, fb3p, n):
  def body(x_ref, w1, b1, w2, b2, w3, b3, o_ref):
    g = jnp.sum(x_ref[...], axis=0, keepdims=True) * (1.0 / n)
    z = jnp.maximum(jnp.dot(g, w1[...], preferred_element_type=jnp.float32)
                    + b1[...], 0.0)
    z = jnp.maximum(jnp.dot(z, w2[...], preferred_element_type=jnp.float32)
                    + b2[...], 0.0)
    z = jax.nn.sigmoid(jnp.dot(z, w3[...], preferred_element_type=jnp.float32)
                       + b3[...])
    o_ref[...] = z

  return pl.pallas_call(
      body,
      out_shape=jax.ShapeDtypeStruct((1, 128), jnp.float32),
  )(x4, fw1, fb1, fw2, fb2, fw3p, fb3p)


def _headmat(a):
  Hh, Cc = a.shape
  M = jnp.zeros((Hh * Cc, 16), jnp.float32)
  rows = jnp.arange(Hh * Cc)
  cols = jnp.repeat(jnp.arange(Hh), Cc)
  return M.at[rows, cols].set(a.reshape(-1))


def kernel(x, edge_index, W1, as1, ad1, b1, W2, as2, ad2, b2, W3, as3, ad3,
           b3, fw1, fb1, fw2, fb2, fw3, fb3):
  N = x.shape[0]
  E = edge_index.shape[1]
  loop = jnp.arange(N, dtype=jnp.int32)
  src = jnp.concatenate([edge_index[0].astype(jnp.int32), loop])
  dst = jnp.concatenate([edge_index[1].astype(jnp.int32), loop])
  sdst, ssrc = lax.sort_key_val(dst, src)

  Etot = E + N
  Npad = ((N + SB - 1) // SB) * SB
  NB = Npad // SB
  Epad = ((Etot + 2 * EC) // EC) * EC
  ssrc = jnp.pad(ssrc, (0, Epad - Etot))
  sdst = jnp.pad(sdst, (0, Epad - Etot), constant_values=Npad)
  bvals = jnp.arange(NB + 1, dtype=jnp.int32) * SB
  bp = jnp.searchsorted(sdst, bvals).astype(jnp.int32)
  bp = jnp.pad(bp, (0, BPL - (NB + 1)), constant_values=Etot)

  xp = jnp.pad(x, ((0, Npad - N), (0, 128 - x.shape[1])))
  W1p = jnp.pad(W1, ((0, 128 - W1.shape[0]), (0, 0)))

  h1, s1, d1, g1 = _tc_layer(xp, W1p, _headmat(as1), _headmat(ad1))
  x2 = _sc_layer(h1, s1, d1, g1[0], b1, ssrc, sdst, bp, 4, 64, N)
  h2, s2, d2, g2 = _tc_layer(x2, W2, _headmat(as2), _headmat(ad2))
  x3 = _sc_layer(h2, s2, d2, g2[0], b2, ssrc, sdst, bp, 4, 64, N)
  h3, s3, d3, g3 = _tc_layer(x3, W3, _headmat(as3), _headmat(ad3))
  x4 = _sc_layer(h3, s3, d3, g3[0], b3, ssrc, sdst, bp, 1, 32, N)

  fw3p = jnp.pad(fw3, ((0, 0), (0, 127)))
  fb3p = jnp.pad(fb3, (0, 127)).reshape(1, 128)
  out = _tc_head(x4, fw1, fb1.reshape(1, -1), fw2, fb2.reshape(1, -1),
                 fw3p, fb3p, N)
  return out[0, 0:1]


# speedup vs baseline: 32.6723x; 1.7018x over previous
import functools

import jax
import jax.numpy as jnp
from jax import lax
from jax.experimental import pallas as pl
from jax.experimental.pallas import tpu as pltpu
from jax.experimental.pallas import tpu_sc as plsc

SB = 256
EC = 64
NW = 32
BPL = 224


def _tc_layer(xp, W, AsP, AdP):
  Npad, Din = xp.shape
  HC = W.shape[1]
  R = 512
  G = Npad // R

  def body(x_ref, w_ref, as_ref, ad_ref, h_ref, asrc_ref, adst_ref, gmax_ref):
    i = pl.program_id(0)
    h = jnp.dot(x_ref[...], w_ref[...], preferred_element_type=jnp.float32)
    h_ref[...] = h
    a_s = jnp.dot(h, as_ref[...], preferred_element_type=jnp.float32)
    a_d = jnp.dot(h, ad_ref[...], preferred_element_type=jnp.float32)
    asrc_ref[...] = a_s
    adst_ref[...] = a_d
    m = jnp.broadcast_to(jnp.max(a_s, axis=0, keepdims=True), (8, 16))

    @pl.when(i == 0)
    def _():
      gmax_ref[...] = m

    @pl.when(i != 0)
    def _():
      gmax_ref[...] = jnp.maximum(gmax_ref[...], m)

  return pl.pallas_call(
      body,
      grid=(G,),
      in_specs=[
          pl.BlockSpec((R, Din), lambda i: (i, 0)),
          pl.BlockSpec((Din, HC), lambda i: (0, 0)),
          pl.BlockSpec((HC, 16), lambda i: (0, 0)),
          pl.BlockSpec((HC, 16), lambda i: (0, 0)),
      ],
      out_specs=[
          pl.BlockSpec((R, HC), lambda i: (i, 0)),
          pl.BlockSpec((R, 16), lambda i: (i, 0)),
          pl.BlockSpec((R, 16), lambda i: (i, 0)),
          pl.BlockSpec((8, 16), lambda i: (0, 0)),
      ],
      out_shape=[
          jax.ShapeDtypeStruct((Npad, HC), jnp.float32),
          jax.ShapeDtypeStruct((Npad, 16), jnp.float32),
          jax.ShapeDtypeStruct((Npad, 16), jnp.float32),
          jax.ShapeDtypeStruct((8, 16), jnp.float32),
      ],
  )(xp, W, AsP, AdP)


def _sc_layer(h, asrcP, adstP, gmax, bias, ssrc, sdst, bp, H, C, n_real):
  HC = H * C
  Npad = h.shape[0]
  NB = Npad // SB
  ITER = (NB + NW - 1) // NW
  mesh = plsc.VectorSubcoreMesh(core_axis_name="c", subcore_axis_name="s")

  @functools.partial(
      pl.kernel,
      mesh=mesh,
      out_type=jax.ShapeDtypeStruct((Npad, HC), jnp.float32),
      compiler_params=pltpu.CompilerParams(use_tc_tiling_on_sc=False),
      scratch_types=[
          pltpu.VMEM((BPL,), jnp.int32),
          pltpu.VMEM((16,), jnp.float32),
          pltpu.VMEM((HC,), jnp.float32),
          pltpu.VMEM((SB, 16), jnp.float32),
          pltpu.VMEM((SB, 16), jnp.float32),
          pltpu.VMEM((SB, HC), jnp.float32),
          pltpu.VMEM((EC,), jnp.int32),
          pltpu.VMEM((EC + 16,), jnp.int32),
          pltpu.VMEM((EC, 16), jnp.float32),
          pltpu.VMEM((EC, HC), jnp.float32),
          pltpu.SemaphoreType.DMA,
      ],
  )
  def k(h_hbm, asrc_hbm, adst_hbm, gmax_hbm, bias_hbm, ssrc_hbm, sdst_hbm,
        bp_hbm, out_hbm, bp_v, gmax_v, bias_v, adst_blk, s_blk, acc,
        src_v, dst_v, asrc_g, h_g, sem):
    wid = lax.axis_index("s") * 2 + lax.axis_index("c")
    pltpu.sync_copy(bp_hbm, bp_v)
    pltpu.sync_copy(gmax_hbm, gmax_v)
    pltpu.sync_copy(bias_hbm, bias_v)
    zf = jnp.zeros((16,), jnp.float32)
    gv = gmax_v[pl.ds(0, 16)]

    def do_block(blk):
      b0 = blk * SB
      bpv16 = bp_v[pl.ds(blk, 16)]
      e_lo = bpv16[0]
      e_hi = bpv16[1]
      abase = (e_lo // 8) * 8
      nch = (e_hi - abase + (EC - 1)) // EC
      pltpu.sync_copy(adst_hbm.at[pl.ds(b0, SB)], adst_blk)

      @plsc.parallel_loop(0, SB, 1, unroll=8)
      def _(i):
        s_blk[i, pl.ds(0, 16)] = zf
        for j in range(HC // 16):
          acc[i, pl.ds(j * 16, 16)] = zf

      def chunk(ci, carry):
        base = abase + ci * EC
        pltpu.sync_copy(ssrc_hbm.at[pl.ds(base, EC)], src_v)
        pltpu.sync_copy(sdst_hbm.at[pl.ds(base, EC)], dst_v.at[pl.ds(0, EC)])
        pltpu.async_copy(asrc_hbm.at[src_v], asrc_g, sem).wait()
        pltpu.async_copy(h_hbm.at[src_v], h_g, sem).wait()

        @plsc.parallel_loop(0, EC, 1, unroll=16)
        def _(e):
          ge = base + e
          vf = jnp.where((ge >= e_lo) & (ge < e_hi), 1.0, 0.0)
          d = dst_v[pl.ds(e, 16)][0]
          dl = jnp.clip(d - b0, 0, SB - 1)
          arow = asrc_g[e, pl.ds(0, 16)]
          drow = adst_blk[dl, pl.ds(0, 16)]
          ev = arow + drow
          ev = jnp.where(ev >= 0, ev, 0.2 * ev)
          cv = gv + drow
          cv = jnp.where(cv >= 0, cv, 0.2 * cv)
          exv = jnp.exp(ev - cv) * vf
          plsc.addupdate(s_blk.at[dl, pl.ds(0, 16)], exv)
          for hh in range(H):
            a = exv[hh]
            for cc in range(C // 16):
              off = hh * C + cc * 16
              v = h_g[e, pl.ds(off, 16)] * a
              plsc.addupdate(acc.at[dl, pl.ds(off, 16)], v)

        return carry

      lax.fori_loop(0, nch, chunk, 0)

      @plsc.parallel_loop(0, SB, 1, unroll=4)
      def _(i):
        rmask = jnp.where(b0 + i < n_real, 1.0, 0.0)
        inv = 1.0 / (s_blk[i, pl.ds(0, 16)] + 1e-16)
        for hh in range(H):
          ivs = inv[hh]
          for cc in range(C // 16):
            off = hh * C + cc * 16
            v = acc[i, pl.ds(off, 16)] * ivs + bias_v[pl.ds(off, 16)]
            v = jnp.where(v > 0, v, jnp.exp(v) - 1.0)
            acc[i, pl.ds(off, 16)] = v * rmask
      pltpu.sync_copy(acc, out_hbm.at[pl.ds(b0, SB)])

    def wbody(it, carry):
      blk = it * NW + wid

      @pl.when(blk < NB)
      def _():
        do_block(blk)

      return carry

    lax.fori_loop(0, ITER, wbody, 0)

  return k(h, asrcP, adstP, gmax, bias, ssrc, sdst, bp)


def _tc_head(x4, fw1, fb1, fw2, fb2, fw3p, fb3p, n):
  def body(x_ref, w1, b1, w2, b2, w3, b3, o_ref):
    g = jnp.sum(x_ref[...], axis=0, keepdims=True) * (1.0 / n)
    z = jnp.maximum(jnp.dot(g, w1[...], preferred_element_type=jnp.float32)
                    + b1[...], 0.0)
    z = jnp.maximum(jnp.dot(z, w2[...], preferred_element_type=jnp.float32)
                    + b2[...], 0.0)
    z = jax.nn.sigmoid(jnp.dot(z, w3[...], preferred_element_type=jnp.float32)
                       + b3[...])
    o_ref[...] = z

  return pl.pallas_call(
      body,
      out_shape=jax.ShapeDtypeStruct((1, 128), jnp.float32),
  )(x4, fw1, fb1, fw2, fb2, fw3p, fb3p)


def _headmat(a):
  Hh, Cc = a.shape
  M = jnp.zeros((Hh * Cc, 16), jnp.float32)
  rows = jnp.arange(Hh * Cc)
  cols = jnp.repeat(jnp.arange(Hh), Cc)
  return M.at[rows, cols].set(a.reshape(-1))


def kernel(x, edge_index, W1, as1, ad1, b1, W2, as2, ad2, b2, W3, as3, ad3,
           b3, fw1, fb1, fw2, fb2, fw3, fb3):
  N = x.shape[0]
  E = edge_index.shape[1]
  loop = jnp.arange(N, dtype=jnp.int32)
  src = jnp.concatenate([edge_index[0].astype(jnp.int32), loop])
  dst = jnp.concatenate([edge_index[1].astype(jnp.int32), loop])
  sdst, ssrc = lax.sort_key_val(dst, src)

  Etot = E + N
  Npad = ((N + SB - 1) // SB) * SB
  NB = Npad // SB
  Epad = ((Etot + 2 * EC) // EC) * EC
  ssrc = jnp.pad(ssrc, (0, Epad - Etot))
  sdst = jnp.pad(sdst, (0, Epad - Etot), constant_values=Npad)
  bvals = jnp.arange(NB + 1, dtype=jnp.int32) * SB
  bp = jnp.searchsorted(sdst, bvals).astype(jnp.int32)
  bp = jnp.pad(bp, (0, BPL - (NB + 1)), constant_values=Etot)

  xp = jnp.pad(x, ((0, Npad - N), (0, 128 - x.shape[1])))
  W1p = jnp.pad(W1, ((0, 128 - W1.shape[0]), (0, 0)))

  h1, s1, d1, g1 = _tc_layer(xp, W1p, _headmat(as1), _headmat(ad1))
  x2 = _sc_layer(h1, s1, d1, g1[0], b1, ssrc, sdst, bp, 4, 64, N)
  h2, s2, d2, g2 = _tc_layer(x2, W2, _headmat(as2), _headmat(ad2))
  x3 = _sc_layer(h2, s2, d2, g2[0], b2, ssrc, sdst, bp, 4, 64, N)
  h3, s3, d3, g3 = _tc_layer(x3, W3, _headmat(as3), _headmat(ad3))
  x4 = _sc_layer(h3, s3, d3, g3[0], b3, ssrc, sdst, bp, 1, 32, N)

  fw3p = jnp.pad(fw3, ((0, 0), (0, 127)))
  fb3p = jnp.pad(fb3, (0, 127)).reshape(1, 128)
  out = _tc_head(x4, fw1, fb1.reshape(1, -1), fw2, fb2.reshape(1, -1),
                 fw3p, fb3p, N)
  return out[0, 0:1]


# speedup vs baseline: 47.0725x; 1.4407x over previous
import functools

import jax
import jax.numpy as jnp
from jax import lax
from jax.experimental import pallas as pl
from jax.experimental.pallas import tpu as pltpu
from jax.experimental.pallas import tpu_sc as plsc

SB = 256
EC = 64
NW = 32
BPL = 224


def _tc_layer(xp, W, AsP, AdP):
  Npad, Din = xp.shape
  HC = W.shape[1]
  R = 512
  G = Npad // R

  def body(x_ref, w_ref, as_ref, ad_ref, h_ref, asrc_ref, adst_ref, gmax_ref):
    i = pl.program_id(0)
    h = jnp.dot(x_ref[...], w_ref[...], preferred_element_type=jnp.float32)
    h_ref[...] = h
    a_s = jnp.dot(h, as_ref[...], preferred_element_type=jnp.float32)
    a_d = jnp.dot(h, ad_ref[...], preferred_element_type=jnp.float32)
    asrc_ref[...] = a_s
    adst_ref[...] = a_d
    m = jnp.broadcast_to(jnp.max(a_s, axis=0, keepdims=True), (8, 16))

    @pl.when(i == 0)
    def _():
      gmax_ref[...] = m

    @pl.when(i != 0)
    def _():
      gmax_ref[...] = jnp.maximum(gmax_ref[...], m)

  return pl.pallas_call(
      body,
      grid=(G,),
      in_specs=[
          pl.BlockSpec((R, Din), lambda i: (i, 0)),
          pl.BlockSpec((Din, HC), lambda i: (0, 0)),
          pl.BlockSpec((HC, 16), lambda i: (0, 0)),
          pl.BlockSpec((HC, 16), lambda i: (0, 0)),
      ],
      out_specs=[
          pl.BlockSpec((R, HC), lambda i: (i, 0)),
          pl.BlockSpec((R, 16), lambda i: (i, 0)),
          pl.BlockSpec((R, 16), lambda i: (i, 0)),
          pl.BlockSpec((8, 16), lambda i: (0, 0)),
      ],
      out_shape=[
          jax.ShapeDtypeStruct((Npad, HC), jnp.float32),
          jax.ShapeDtypeStruct((Npad, 16), jnp.float32),
          jax.ShapeDtypeStruct((Npad, 16), jnp.float32),
          jax.ShapeDtypeStruct((8, 16), jnp.float32),
      ],
  )(xp, W, AsP, AdP)


def _sc_layer(h, asrcP, adstP, gmax, bias, ssrc, sdst, bp, H, C, n_real):
  HC = H * C
  Npad = h.shape[0]
  NB = Npad // SB
  ITER = (NB + NW - 1) // NW
  mesh = plsc.VectorSubcoreMesh(core_axis_name="c", subcore_axis_name="s")

  @functools.partial(
      pl.kernel,
      mesh=mesh,
      out_type=jax.ShapeDtypeStruct((Npad, HC), jnp.float32),
      compiler_params=pltpu.CompilerParams(use_tc_tiling_on_sc=False),
      scratch_types=[
          pltpu.VMEM((BPL,), jnp.int32),
          pltpu.VMEM((16,), jnp.float32),
          pltpu.VMEM((HC,), jnp.float32),
          pltpu.VMEM((SB, 16), jnp.float32),
          pltpu.VMEM((SB, 16), jnp.float32),
          pltpu.VMEM((SB, HC), jnp.float32),
          pltpu.VMEM((EC,), jnp.int32),
          pltpu.VMEM((EC + 16,), jnp.int32),
          pltpu.VMEM((EC, 16), jnp.float32),
          pltpu.VMEM((EC, HC), jnp.float32),
          pltpu.VMEM((EC,), jnp.int32),
          pltpu.VMEM((EC + 16,), jnp.int32),
          pltpu.VMEM((EC, 16), jnp.float32),
          pltpu.VMEM((EC, HC), jnp.float32),
          pltpu.SemaphoreType.DMA,
          pltpu.SemaphoreType.DMA,
          pltpu.SemaphoreType.DMA,
          pltpu.SemaphoreType.DMA,
      ],
  )
  def k(h_hbm, asrc_hbm, adst_hbm, gmax_hbm, bias_hbm, ssrc_hbm, sdst_hbm,
        bp_hbm, out_hbm, bp_v, gmax_v, bias_v, adst_blk, s_blk, acc,
        src_v0, dst_v0, asrc_g0, h_g0, src_v1, dst_v1, asrc_g1, h_g1,
        sem_a0, sem_h0, sem_a1, sem_h1):
    wid = lax.axis_index("s") * 2 + lax.axis_index("c")
    pltpu.sync_copy(bp_hbm, bp_v)
    pltpu.sync_copy(gmax_hbm, gmax_v)
    pltpu.sync_copy(bias_hbm, bias_v)
    zf = jnp.zeros((16,), jnp.float32)
    gv = gmax_v[pl.ds(0, 16)]

    def do_block(blk):
      b0 = blk * SB
      bpv16 = bp_v[pl.ds(blk, 16)]
      e_lo = bpv16[0]
      e_hi = bpv16[1]
      abase = (e_lo // 8) * 8
      nch = (e_hi - abase + (EC - 1)) // EC
      pltpu.sync_copy(adst_hbm.at[pl.ds(b0, SB)], adst_blk)

      @plsc.parallel_loop(0, SB, 1, unroll=8)
      def _(i):
        s_blk[i, pl.ds(0, 16)] = zf
        for j in range(HC // 16):
          acc[i, pl.ds(j * 16, 16)] = zf

      slots = ((src_v0, dst_v0, asrc_g0, h_g0, sem_a0, sem_h0),
               (src_v1, dst_v1, asrc_g1, h_g1, sem_a1, sem_h1))

      def fetch(ci, slot):
        srcv, dstv, ag, hg, sa, sh = slot
        base = abase + ci * EC
        pltpu.sync_copy(ssrc_hbm.at[pl.ds(base, EC)], srcv)
        pltpu.sync_copy(sdst_hbm.at[pl.ds(base, EC)], dstv.at[pl.ds(0, EC)])
        pltpu.async_copy(asrc_hbm.at[srcv], ag, sa)
        pltpu.async_copy(h_hbm.at[srcv], hg, sh)

      def drain(slot):
        srcv, dstv, ag, hg, sa, sh = slot
        pltpu.make_async_copy(asrc_hbm.at[srcv], ag, sa).wait()
        pltpu.make_async_copy(h_hbm.at[srcv], hg, sh).wait()

      def process(ci, slot):
        srcv, dstv, ag, hg, sa, sh = slot
        base = abase + ci * EC

        @plsc.parallel_loop(0, EC, 1, unroll=16)
        def _(e):
          ge = base + e
          vf = jnp.where((ge >= e_lo) & (ge < e_hi), 1.0, 0.0)
          d = dstv[pl.ds(e, 16)][0]
          dl = jnp.clip(d - b0, 0, SB - 1)
          arow = ag[e, pl.ds(0, 16)]
          drow = adst_blk[dl, pl.ds(0, 16)]
          ev = arow + drow
          ev = jnp.where(ev >= 0, ev, 0.2 * ev)
          cv = gv + drow
          cv = jnp.where(cv >= 0, cv, 0.2 * cv)
          exv = jnp.exp(ev - cv) * vf
          plsc.addupdate(s_blk.at[dl, pl.ds(0, 16)], exv)
          for hh in range(H):
            a = exv[hh]
            for cc in range(C // 16):
              off = hh * C + cc * 16
              v = hg[e, pl.ds(off, 16)] * a
              plsc.addupdate(acc.at[dl, pl.ds(off, 16)], v)

      @pl.when(nch > 0)
      def _():
        fetch(0, slots[0])

      def pair(po, carry):
        for s in range(2):
          ci = po * 2 + s

          @pl.when(ci < nch)
          def _(ci=ci, s=s):
            @pl.when(ci + 1 < nch)
            def _():
              fetch(ci + 1, slots[1 - s])

            drain(slots[s])
            process(ci, slots[s])

        return carry

      lax.fori_loop(0, (nch + 1) // 2, pair, 0)

      @plsc.parallel_loop(0, SB, 1, unroll=4)
      def _(i):
        rmask = jnp.where(b0 + i < n_real, 1.0, 0.0)
        inv = 1.0 / (s_blk[i, pl.ds(0, 16)] + 1e-16)
        for hh in range(H):
          ivs = inv[hh]
          for cc in range(C // 16):
            off = hh * C + cc * 16
            v = acc[i, pl.ds(off, 16)] * ivs + bias_v[pl.ds(off, 16)]
            v = jnp.where(v > 0, v, jnp.exp(v) - 1.0)
            acc[i, pl.ds(off, 16)] = v * rmask
      pltpu.sync_copy(acc, out_hbm.at[pl.ds(b0, SB)])

    def wbody(it, carry):
      blk = it * NW + wid

      @pl.when(blk < NB)
      def _():
        do_block(blk)

      return carry

    lax.fori_loop(0, ITER, wbody, 0)

  return k(h, asrcP, adstP, gmax, bias, ssrc, sdst, bp)


def _tc_head(x4, fw1, fb1, fw2, fb2, fw3p, fb3p, n):
  def body(x_ref, w1, b1, w2, b2, w3, b3, o_ref):
    g = jnp.sum(x_ref[...], axis=0, keepdims=True) * (1.0 / n)
    z = jnp.maximum(jnp.dot(g, w1[...], preferred_element_type=jnp.float32)
                    + b1[...], 0.0)
    z = jnp.maximum(jnp.dot(z, w2[...], preferred_element_type=jnp.float32)
                    + b2[...], 0.0)
    z = jax.nn.sigmoid(jnp.dot(z, w3[...], preferred_element_type=jnp.float32)
                       + b3[...])
    o_ref[...] = z

  return pl.pallas_call(
      body,
      out_shape=jax.ShapeDtypeStruct((1, 128), jnp.float32),
  )(x4, fw1, fb1, fw2, fb2, fw3p, fb3p)


def _headmat(a):
  Hh, Cc = a.shape
  M = jnp.zeros((Hh * Cc, 16), jnp.float32)
  rows = jnp.arange(Hh * Cc)
  cols = jnp.repeat(jnp.arange(Hh), Cc)
  return M.at[rows, cols].set(a.reshape(-1))


def kernel(x, edge_index, W1, as1, ad1, b1, W2, as2, ad2, b2, W3, as3, ad3,
           b3, fw1, fb1, fw2, fb2, fw3, fb3):
  N = x.shape[0]
  E = edge_index.shape[1]
  loop = jnp.arange(N, dtype=jnp.int32)
  src = jnp.concatenate([edge_index[0].astype(jnp.int32), loop])
  dst = jnp.concatenate([edge_index[1].astype(jnp.int32), loop])
  sdst, ssrc = lax.sort_key_val(dst, src)

  Etot = E + N
  Npad = ((N + SB - 1) // SB) * SB
  NB = Npad // SB
  Epad = ((Etot + 2 * EC) // EC) * EC
  ssrc = jnp.pad(ssrc, (0, Epad - Etot))
  sdst = jnp.pad(sdst, (0, Epad - Etot), constant_values=Npad)
  bvals = jnp.arange(NB + 1, dtype=jnp.int32) * SB
  bp = jnp.searchsorted(sdst, bvals).astype(jnp.int32)
  bp = jnp.pad(bp, (0, BPL - (NB + 1)), constant_values=Etot)

  xp = jnp.pad(x, ((0, Npad - N), (0, 128 - x.shape[1])))
  W1p = jnp.pad(W1, ((0, 128 - W1.shape[0]), (0, 0)))

  h1, s1, d1, g1 = _tc_layer(xp, W1p, _headmat(as1), _headmat(ad1))
  x2 = _sc_layer(h1, s1, d1, g1[0], b1, ssrc, sdst, bp, 4, 64, N)
  h2, s2, d2, g2 = _tc_layer(x2, W2, _headmat(as2), _headmat(ad2))
  x3 = _sc_layer(h2, s2, d2, g2[0], b2, ssrc, sdst, bp, 4, 64, N)
  h3, s3, d3, g3 = _tc_layer(x3, W3, _headmat(as3), _headmat(ad3))
  x4 = _sc_layer(h3, s3, d3, g3[0], b3, ssrc, sdst, bp, 1, 32, N)

  fw3p = jnp.pad(fw3, ((0, 0), (0, 127)))
  fb3p = jnp.pad(fb3, (0, 127)).reshape(1, 128)
  out = _tc_head(x4, fw1, fb1.reshape(1, -1), fw2, fb2.reshape(1, -1),
                 fw3p, fb3p, N)
  return out[0, 0:1]
